# baseline (device time: 188021 ns/iter reference)
import jax
import jax.numpy as jnp
from jax import lax
from jax.experimental import pallas as pl
from jax.experimental.pallas import tpu as pltpu

N_DEV = 8
B_CH = 64
D = 2048
H_LOC = 4096
K_T = 512
N_K = H_LOC // K_T


def _layer(x_chunk, Win, Wout, cid):

    def body(x_ref, win_ref, wout_ref, out_ref,
             xg_ref, p_ref, psend_ref, prec_ref,
             ag_send, ag_recv, rs_send, rs_recv):
        k = pl.program_id(0)
        my = lax.axis_index("i")

        @pl.when(k == 0)
        def _allgather():
            bsem = pltpu.get_barrier_semaphore()
            for o in range(1, N_DEV):
                pl.semaphore_signal(
                    bsem, inc=1,
                    device_id=((my + o) % N_DEV,),
                    device_id_type=pl.DeviceIdType.MESH,
                )
            pl.semaphore_wait(bsem, N_DEV - 1)

            xg_ref[my] = x_ref[...].astype(jnp.bfloat16)

            sends = []
            for o in range(1, N_DEV):
                rdma = pltpu.make_async_remote_copy(
                    src_ref=xg_ref.at[my],
                    dst_ref=xg_ref.at[my],
                    send_sem=ag_send.at[o],
                    recv_sem=ag_recv.at[o],
                    device_id=((my + o) % N_DEV,),
                    device_id_type=pl.DeviceIdType.MESH,
                )
                rdma.start()
                sends.append(rdma)
            for o in range(1, N_DEV):
                pltpu.make_async_remote_copy(
                    src_ref=xg_ref.at[0],
                    dst_ref=xg_ref.at[0],
                    send_sem=ag_send.at[o],
                    recv_sem=ag_recv.at[o],
                    device_id=(0,),
                    device_id_type=pl.DeviceIdType.MESH,
                ).wait_recv()
            for rdma in sends:
                rdma.wait_send()
            p_ref[...] = jnp.zeros_like(p_ref)

        xg = xg_ref[...].reshape(N_DEV * B_CH, D)
        wb = win_ref[...].astype(jnp.bfloat16)
        h = jnp.dot(xg, wb, preferred_element_type=jnp.float32)
        hb = jnp.maximum(h, 0.0).astype(jnp.bfloat16)
        wo = wout_ref[...].astype(jnp.bfloat16)
        p_ref[...] += jnp.dot(hb, wo, preferred_element_type=jnp.float32)

        @pl.when(k == N_K - 1)
        def _reducescatter():
            psend_ref[...] = (
                p_ref[...].reshape(N_DEV, B_CH, D).astype(jnp.bfloat16)
            )
            prec_ref[my] = psend_ref[my]
            sends = []
            for o in range(1, N_DEV):
                rdma = pltpu.make_async_remote_copy(
                    src_ref=psend_ref.at[(my + o) % N_DEV],
                    dst_ref=prec_ref.at[my],
                    send_sem=rs_send.at[o],
                    recv_sem=rs_recv.at[o],
                    device_id=((my + o) % N_DEV,),
                    device_id_type=pl.DeviceIdType.MESH,
                )
                rdma.start()
                sends.append(rdma)
            for o in range(1, N_DEV):
                pltpu.make_async_remote_copy(
                    src_ref=prec_ref.at[0],
                    dst_ref=prec_ref.at[0],
                    send_sem=rs_send.at[o],
                    recv_sem=rs_recv.at[o],
                    device_id=(0,),
                    device_id_type=pl.DeviceIdType.MESH,
                ).wait_recv()
            for rdma in sends:
                rdma.wait_send()
            acc = prec_ref[0].astype(jnp.float32)
            for j in range(1, N_DEV):
                acc = acc + prec_ref[j].astype(jnp.float32)
            out_ref[...] = acc

    return pl.pallas_call(
        body,
        grid=(N_K,),
        in_specs=[
            pl.BlockSpec((B_CH, D), lambda k: (0, 0)),
            pl.BlockSpec((D, K_T), lambda k: (0, k)),
            pl.BlockSpec((K_T, D), lambda k: (k, 0)),
        ],
        out_specs=pl.BlockSpec((B_CH, D), lambda k: (0, 0)),
        out_shape=jax.ShapeDtypeStruct((B_CH, D), jnp.float32),
        scratch_shapes=[
            pltpu.VMEM((N_DEV, B_CH, D), jnp.bfloat16),
            pltpu.VMEM((N_DEV * B_CH, D), jnp.float32),
            pltpu.VMEM((N_DEV, B_CH, D), jnp.bfloat16),
            pltpu.VMEM((N_DEV, B_CH, D), jnp.bfloat16),
            pltpu.SemaphoreType.DMA((N_DEV,)),
            pltpu.SemaphoreType.DMA((N_DEV,)),
            pltpu.SemaphoreType.DMA((N_DEV,)),
            pltpu.SemaphoreType.DMA((N_DEV,)),
        ],
        compiler_params=pltpu.CompilerParams(
            collective_id=cid,
            dimension_semantics=("arbitrary",),
        ),
    )(x_chunk, Win, Wout)


def kernel(x, Win0, Wout0, Win1, Wout1, Win2, Wout2):
    x = _layer(x, Win0, Wout0, 0)
    x = _layer(x, Win1, Wout1, 1)
    x = _layer(x, Win2, Wout2, 2)
    return x
